# baseline (speedup 1.0000x reference)
import jax
import jax.numpy as jnp
from jax.experimental import pallas as pl
from jax.experimental.pallas import tpu as pltpu

_NCHUNK = 256
_TBLK = 4
_APAD = 264


def _vq_kernel(z_ref, w_ref, out_ref, wa_ref, za_ref):
    @pl.when(pl.program_id(0) == 0)
    def _():
        w = w_ref[...]
        wa_ref[:, 0:256] = -(w + w)
        wa_ref[:, 256:257] = jnp.sum(w * w, axis=1, keepdims=True)
        wa_ref[:, 257:] = jnp.zeros_like(wa_ref[:, 257:])
        za_ref[256:257, :] = jnp.ones_like(za_ref[256:257, :])
        za_ref[257:, :] = jnp.zeros_like(za_ref[257:, :])

    wa = wa_ref[...]
    n = z_ref.shape[2]
    for tt in range(z_ref.shape[0]):
        za_ref[0:256, :] = z_ref[tt]
        za = za_ref[...]
        for c in range(0, n, _NCHUNK):
            key = jax.lax.dot_general(
                wa, za[:, c:c + _NCHUNK], (((1,), (0,)), ((), ())),
                preferred_element_type=jnp.float32,
                precision=jax.lax.Precision.DEFAULT)
            out_ref[tt, 0, c:c + _NCHUNK] = jnp.argmin(key, axis=0).astype(jnp.int32)


def kernel(z, W):
    t, a, b, c = z.shape
    n = b * c
    k = W.shape[0]
    z3 = z.reshape(t, a, n)
    out = pl.pallas_call(
        _vq_kernel,
        grid=(t // _TBLK,),
        in_specs=[
            pl.BlockSpec((_TBLK, a, n), lambda i: (i, 0, 0)),
            pl.BlockSpec((k, a), lambda i: (0, 0)),
        ],
        out_specs=pl.BlockSpec((_TBLK, 1, n), lambda i: (i, 0, 0)),
        out_shape=jax.ShapeDtypeStruct((t, 1, n), jnp.int32),
        scratch_shapes=[
            pltpu.VMEM((k, _APAD), jnp.float32),
            pltpu.VMEM((_APAD, n), jnp.float32),
        ],
    )(z3, W)
    return out.reshape(t, b, c)
